# baseline (speedup 1.0000x reference)
import functools

import jax
import jax.numpy as jnp
import numpy as np
from jax import lax
from jax.experimental import pallas as pl
from jax.experimental.pallas import tpu as pltpu
from jax.experimental.pallas import tpu_sc as plsc

LANES = 128
CHUNK = 512
KPOS_MAX = 128
BATCH = 512
TOPBIT = np.uint32(0x80000000)
LOWMASK = np.uint32(0x7FFFFFFF)
NBINS = 2048
SC_CH = 4096


def _sc_body(logits_hbm, targets_hbm, out_hbm, xbuf, tbuf, hist, red,
             mbuf, tmpv, shared2):
    cid = lax.axis_index("c")
    sid = lax.axis_index("s")
    per_tile = logits_hbm.shape[0] // 16
    nch = per_tile // SC_CH
    iota = lax.iota(jnp.int32, 16)
    zero16 = jnp.zeros((16,), jnp.int32)
    ones16 = jnp.full((16,), 1, jnp.int32)
    is_c0 = cid == 0
    c0v = jnp.broadcast_to(is_c0, (16,))
    flip = jnp.where(c0v, jnp.full((16,), np.uint32(0xFFFFFFFF), jnp.uint32),
                     jnp.zeros((16,), jnp.uint32))
    lanebase = iota * NBINS

    def zero_hist():
        def zh(i, _):
            plsc.store_scatter(hist, [i * 16 + iota], zero16)
            return 0
        lax.fori_loop(0, (NBINS * 16) // 16, zh, 0)

    zero_hist()

    def stream_pass(binfn):
        def chunk_body(ci, acc):
            base = sid * per_tile + ci * SC_CH
            pltpu.sync_copy(logits_hbm.at[pl.ds(base, SC_CH)], xbuf)
            pltpu.sync_copy(targets_hbm.at[pl.ds(base, SC_CH)], tbuf)

            def vec_body(v, acc2):
                off = v * 16
                x = xbuf[pl.ds(off, 16)]
                t = tbuf[pl.ds(off, 16)]
                b = lax.bitcast_convert_type(x, jnp.uint32)
                neg = b >= TOPBIT
                ukey = jnp.where(neg, ~b, b | TOPBIT)
                key = ukey ^ flip
                sel = (t >= 1) == c0v
                keym = jnp.where(sel, key, np.uint32(0))
                bn, m = binfn(keym, sel)
                plsc.addupdate_scatter(hist, [lanebase + bn], ones16, mask=m)
                return acc2 + jnp.where(m, 1, 0)
            return lax.fori_loop(0, SC_CH // 16, vec_body, acc)
        return jnp.sum(lax.fori_loop(0, nch, chunk_body, zero16))

    def bin_l1(keym, sel):
        return lax.bitcast_convert_type(keym >> np.uint32(21), jnp.int32), sel
    selcnt = stream_pass(bin_l1)

    def reduce_hist(nrows):
        def rb(r, _):
            rbase = r * 16
            acc = zero16
            for l in range(16):
                acc = acc + plsc.load_gather(hist, [l * NBINS + rbase + iota])
            red[r] = acc
            return 0
        lax.fori_loop(0, nrows, rb, 0)

    def merge_and_fetch(nrows):
        pltpu.sync_copy(red, shared2.at[sid])
        plsc.subcore_barrier()
        pltpu.sync_copy(shared2.at[0], red)

        def mj(j, _):
            pltpu.sync_copy(shared2.at[j], mbuf)

            def rr(r, _2):
                red[r] = red[r] + mbuf[r]
                return 0
            lax.fori_loop(0, nrows, rr, 0)
            return 0
        lax.fori_loop(1, 16, mj, 0)
        plsc.subcore_barrier()

    def scan_rows(nrows, k):
        def sb(i, carry):
            cum, row_sel, cum_above, row_keep = carry
            r = nrows - 1 - i
            row = red[r]
            s = jnp.sum(row)
            hit = (row_sel < 0) & (cum + s >= k) & (k > 0)
            hitv = jnp.broadcast_to(hit, (16,))
            row_sel = jnp.where(hit, r, row_sel)
            cum_above = jnp.where(hit, cum, cum_above)
            row_keep = jnp.where(hitv, row, row_keep)
            return (cum + s, row_sel, cum_above, row_keep)

        cum, row_sel, cum_above, row_keep = lax.fori_loop(
            0, nrows, sb, (0, -1, 0, zero16))
        rv = jnp.flip(row_keep, 0)
        c = jnp.cumsum(rv)
        kr = k - cum_above
        j = jnp.min(jnp.where(c >= kr, iota, 16))
        jv = jnp.broadcast_to(j, (16,))
        cj = jnp.sum(jnp.where(iota == jv, c, 0))
        rj = jnp.sum(jnp.where(iota == jv, rv, 0))
        b = row_sel * 16 + (15 - j)
        above = cum_above + cj - rj
        return b, above, cum

    def refine_pass(mshift, mprefix, bshift, bmask, nrows):
        zero_hist()
        pm = jnp.broadcast_to(mprefix, (16,))

        def bin_ref(keym, sel):
            match = (keym >> mshift) == pm
            bn = lax.bitcast_convert_type((keym >> bshift) & bmask, jnp.int32)
            return bn, match
        stream_pass(bin_ref)
        reduce_hist(nrows)
        merge_and_fetch(nrows)

    reduce_hist(128)

    def loc_body(r, acc):
        return acc + jnp.sum(red[r])
    local_tot = lax.fori_loop(0, 128, loc_body, 0)
    merge_and_fetch(128)
    n_total = per_tile * 16
    def tot_body(r, acc):
        return acc + jnp.sum(red[r])
    own_total = lax.fori_loop(0, 128, tot_body, 0)
    cpos = jnp.where(is_c0, own_total, n_total - own_total)
    cneg = n_total - cpos
    k1 = jnp.minimum(cpos, KPOS_MAX)
    k2 = jnp.minimum(BATCH - k1, cneg)
    k = jnp.where(is_c0, k1, k2)

    b1, a1, _ = scan_rows(128, k)
    p1u = b1.astype(jnp.uint32)

    refine_pass(np.uint32(21), p1u, np.uint32(10), np.uint32(0x7FF), 128)
    b2, a2, _ = scan_rows(128, k - a1)
    p2u = (p1u << np.uint32(11)) | b2.astype(jnp.uint32)

    refine_pass(np.uint32(10), p2u, np.uint32(0), np.uint32(0x3FF), 64)
    b3, a3, _ = scan_rows(64, k - a1 - a2)

    t_key = (p2u << np.uint32(10)) | b3.astype(jnp.uint32)
    t_key = jnp.where(k > 0, t_key, np.uint32(0xFFFFFFFF))
    r_tie = jnp.where(k > 0, k - (a1 + a2 + a3), 0)

    tbits = lax.bitcast_convert_type(jnp.broadcast_to(t_key, (16,)), jnp.int32)
    outv = jnp.where(iota == 0, tbits, zero16)
    outv = jnp.where(iota == 1, jnp.broadcast_to(r_tie, (16,)), outv)
    outv = jnp.where(iota == 2, jnp.broadcast_to(cpos, (16,)), outv)
    outv = jnp.where(iota == 3, jnp.broadcast_to(cneg, (16,)), outv)
    outv = jnp.where(iota == 4, jnp.broadcast_to(selcnt, (16,)), outv)
    outv = jnp.where(iota == 5, jnp.broadcast_to(local_tot, (16,)), outv)

    @pl.when(sid == 0)
    def _():
        tmpv[...] = outv
        pltpu.sync_copy(tmpv, out_hbm.at[cid])


def _softplus(t):
    return jnp.maximum(t, 0.0) + jnp.log1p(jnp.exp(-jnp.abs(t)))


def _unmono(ukey):
    b = jnp.where(ukey >= TOPBIT, ukey & LOWMASK, ~ukey)
    return lax.bitcast_convert_type(b, jnp.float32)


def _tc_body(params_ref, logits_ref, targets_ref, out_ref):
    rows = logits_ref.shape[0]
    nchunk = rows // CHUNK
    tpos = lax.bitcast_convert_type(params_ref[0, 0], jnp.uint32)
    rpos = params_ref[0, 1]
    tneg = lax.bitcast_convert_type(params_ref[1, 0], jnp.uint32)
    rneg = params_ref[1, 1]

    def fin_chunk(i, carry):
        s1, s2 = carry
        x = logits_ref[pl.ds(i * CHUNK, CHUNK), :]
        t = targets_ref[pl.ds(i * CHUNK, CHUNK), :]
        b = lax.bitcast_convert_type(x, jnp.uint32)
        ukey = jnp.where(b >= TOPBIT, ~b, b | TOPBIT)
        kpos = jnp.where(t >= 1, ~ukey, np.uint32(0))
        kneg = jnp.where(t == 0, ukey, np.uint32(0))
        l1 = jnp.minimum(_softplus(-x), 100.0)
        l2 = jnp.minimum(_softplus(x), 100.0)
        s1 = s1 + jnp.sum(jnp.where(kpos > tpos, l1, 0.0))
        s2 = s2 + jnp.sum(jnp.where(kneg > tneg, l2, 0.0))
        return s1, s2

    s1, s2 = lax.fori_loop(0, nchunk, fin_chunk, (0.0, 0.0))
    tl1v = jnp.minimum(_softplus(-_unmono(jnp.full((8, 128), ~tpos))), 100.0)
    tl2v = jnp.minimum(_softplus(_unmono(jnp.full((8, 128), tneg))), 100.0)
    i0 = lax.broadcasted_iota(jnp.int32, (8, 128), 0)
    i1 = lax.broadcasted_iota(jnp.int32, (8, 128), 1)
    m00 = (i0 == 0) & (i1 == 0)
    s1 = s1 + jnp.sum(jnp.where(m00 & (rpos > 0),
                                rpos.astype(jnp.float32) * tl1v, 0.0))
    s2 = s2 + jnp.sum(jnp.where(m00 & (rneg > 0),
                                rneg.astype(jnp.float32) * tl2v, 0.0))
    out_ref[...] = jnp.broadcast_to((s1 + s2) / float(BATCH), (1, 1))


N_TOTAL = 1048576
PER_TILE = N_TOTAL // 16


def _make_sck():
    mesh = plsc.VectorSubcoreMesh(core_axis_name="c", subcore_axis_name="s")
    return pl.kernel(
        _sc_body,
        mesh=mesh,
        out_type=jax.ShapeDtypeStruct((2, 16), jnp.int32),
        compiler_params=pltpu.CompilerParams(needs_layout_passes=False),
        scratch_types=[
            pltpu.VMEM((SC_CH,), jnp.float32),
            pltpu.VMEM((SC_CH,), jnp.int32),
            pltpu.VMEM((NBINS * 16,), jnp.int32),
            pltpu.VMEM((128, 16), jnp.int32),
            pltpu.VMEM((128, 16), jnp.int32),
            pltpu.VMEM((16,), jnp.int32),
            pltpu.VMEM_SHARED((16, 128, 16), jnp.int32),
        ],
    )


_SCK = _make_sck()


def kernel(logits, targets):
    n = logits.shape[0]
    flat = logits.reshape(n)
    params = _SCK(flat, targets)

    rows = n // LANES
    out = pl.pallas_call(
        _tc_body,
        out_shape=jax.ShapeDtypeStruct((1, 1), jnp.float32),
        in_specs=[
            pl.BlockSpec(memory_space=pltpu.SMEM),
            pl.BlockSpec(memory_space=pltpu.VMEM),
            pl.BlockSpec(memory_space=pltpu.VMEM),
        ],
        out_specs=pl.BlockSpec(memory_space=pltpu.VMEM),
    )(params, flat.reshape(rows, LANES), targets.reshape(rows, LANES))
    return out[0, 0]


# speedup vs baseline: 1.1487x; 1.1487x over previous
import functools

import jax
import jax.numpy as jnp
import numpy as np
from jax import lax
from jax.experimental import pallas as pl
from jax.experimental.pallas import tpu as pltpu
from jax.experimental.pallas import tpu_sc as plsc

LANES = 128
CHUNK = 512
KPOS_MAX = 128
BATCH = 512
TOPBIT = np.uint32(0x80000000)
LOWMASK = np.uint32(0x7FFFFFFF)
NBINS = 2048
SC_CH = 4096


def _sc_body(logits_hbm, targets_hbm, out_hbm, xbuf, tbuf, hist, red,
             mbuf, tmpv, shared2):
    cid = lax.axis_index("c")
    sid = lax.axis_index("s")
    per_tile = logits_hbm.shape[0] // 16
    nch = per_tile // SC_CH
    iota = lax.iota(jnp.int32, 16)
    zero16 = jnp.zeros((16,), jnp.int32)
    ones16 = jnp.full((16,), 1, jnp.int32)
    is_c0 = cid == 0
    c0v = jnp.broadcast_to(is_c0, (16,))
    flip = jnp.where(c0v, jnp.full((16,), np.uint32(0xFFFFFFFF), jnp.uint32),
                     jnp.zeros((16,), jnp.uint32))
    lanebase = iota * NBINS

    def zero_hist():
        def zh(i, _):
            for u in range(4):
                plsc.store_scatter(hist, [(i * 4 + u) * 16 + iota], zero16)
            return 0
        lax.fori_loop(0, (NBINS * 16) // 64, zh, 0)

    zero_hist()

    def stream_pass(binfn):
        def chunk_body(ci, _):
            base = sid * per_tile + ci * SC_CH
            pltpu.sync_copy(logits_hbm.at[pl.ds(base, SC_CH)], xbuf)
            pltpu.sync_copy(targets_hbm.at[pl.ds(base, SC_CH)], tbuf)

            def vec_body(v, _c):
                for u in range(4):
                    off = (v * 4 + u) * 16
                    x = xbuf[pl.ds(off, 16)]
                    t = tbuf[pl.ds(off, 16)]
                    b = lax.bitcast_convert_type(x, jnp.uint32)
                    neg = b >= TOPBIT
                    ukey = jnp.where(neg, ~b, b | TOPBIT)
                    key = ukey ^ flip
                    sel = (t >= 1) == c0v
                    keym = jnp.where(sel, key, np.uint32(0))
                    bn, m = binfn(keym, sel)
                    plsc.addupdate_scatter(hist, [lanebase + bn], ones16,
                                           mask=m)
                return 0
            lax.fori_loop(0, SC_CH // 64, vec_body, 0)
            return 0
        lax.fori_loop(0, nch, chunk_body, 0)

    def bin_l1(keym, sel):
        return lax.bitcast_convert_type(keym >> np.uint32(21), jnp.int32), sel
    stream_pass(bin_l1)

    def reduce_hist(nrows):
        def rb(r, _):
            rbase = r * 16
            acc = zero16
            for l in range(16):
                acc = acc + plsc.load_gather(hist, [l * NBINS + rbase + iota])
            red[r] = acc
            return 0
        lax.fori_loop(0, nrows, rb, 0)

    def merge_and_fetch(nrows):
        pltpu.sync_copy(red, shared2.at[sid])
        plsc.subcore_barrier()
        pltpu.sync_copy(shared2.at[0], red)

        def mj(j, _):
            pltpu.sync_copy(shared2.at[j], mbuf)

            def rr(r, _2):
                for u in range(4):
                    rw = r * 4 + u
                    red[rw] = red[rw] + mbuf[rw]
                return 0
            lax.fori_loop(0, nrows // 4, rr, 0)
            return 0
        lax.fori_loop(1, 16, mj, 0)
        plsc.subcore_barrier()

    def scan_rows(nrows, k):
        def sb(i, carry):
            cum, row_sel, cum_above, row_keep = carry
            r = nrows - 1 - i
            row = red[r]
            s = jnp.sum(row)
            hit = (row_sel < 0) & (cum + s >= k) & (k > 0)
            hitv = jnp.broadcast_to(hit, (16,))
            row_sel = jnp.where(hit, r, row_sel)
            cum_above = jnp.where(hit, cum, cum_above)
            row_keep = jnp.where(hitv, row, row_keep)
            return (cum + s, row_sel, cum_above, row_keep)

        cum, row_sel, cum_above, row_keep = lax.fori_loop(
            0, nrows, sb, (0, -1, 0, zero16))
        rv = jnp.flip(row_keep, 0)
        c = jnp.cumsum(rv)
        kr = k - cum_above
        j = jnp.min(jnp.where(c >= kr, iota, 16))
        jv = jnp.broadcast_to(j, (16,))
        cj = jnp.sum(jnp.where(iota == jv, c, 0))
        rj = jnp.sum(jnp.where(iota == jv, rv, 0))
        b = row_sel * 16 + (15 - j)
        above = cum_above + cj - rj
        return b, above, cum

    def refine_pass(mshift, mprefix, bshift, bmask, nrows):
        zero_hist()
        pm = jnp.broadcast_to(mprefix, (16,))

        def bin_ref(keym, sel):
            match = (keym >> mshift) == pm
            bn = lax.bitcast_convert_type((keym >> bshift) & bmask, jnp.int32)
            return bn, match
        stream_pass(bin_ref)
        reduce_hist(nrows)
        merge_and_fetch(nrows)

    reduce_hist(128)
    merge_and_fetch(128)
    n_total = per_tile * 16
    def tot_body(r, acc):
        return acc + jnp.sum(red[r])
    own_total = lax.fori_loop(0, 128, tot_body, 0)
    cpos = jnp.where(is_c0, own_total, n_total - own_total)
    cneg = n_total - cpos
    k1 = jnp.minimum(cpos, KPOS_MAX)
    k2 = jnp.minimum(BATCH - k1, cneg)
    k = jnp.where(is_c0, k1, k2)

    b1, a1, _ = scan_rows(128, k)
    p1u = b1.astype(jnp.uint32)

    refine_pass(np.uint32(21), p1u, np.uint32(10), np.uint32(0x7FF), 128)
    b2, a2, _ = scan_rows(128, k - a1)
    p2u = (p1u << np.uint32(11)) | b2.astype(jnp.uint32)

    refine_pass(np.uint32(10), p2u, np.uint32(0), np.uint32(0x3FF), 64)
    b3, a3, _ = scan_rows(64, k - a1 - a2)

    t_key = (p2u << np.uint32(10)) | b3.astype(jnp.uint32)
    t_key = jnp.where(k > 0, t_key, np.uint32(0xFFFFFFFF))
    r_tie = jnp.where(k > 0, k - (a1 + a2 + a3), 0)

    tbits = lax.bitcast_convert_type(jnp.broadcast_to(t_key, (16,)), jnp.int32)
    outv = jnp.where(iota == 0, tbits, zero16)
    outv = jnp.where(iota == 1, jnp.broadcast_to(r_tie, (16,)), outv)
    outv = jnp.where(iota == 2, jnp.broadcast_to(cpos, (16,)), outv)
    outv = jnp.where(iota == 3, jnp.broadcast_to(cneg, (16,)), outv)

    @pl.when(sid == 0)
    def _():
        tmpv[...] = outv
        pltpu.sync_copy(tmpv, out_hbm.at[cid])


def _softplus(t):
    return jnp.maximum(t, 0.0) + jnp.log1p(jnp.exp(-jnp.abs(t)))


def _unmono(ukey):
    b = jnp.where(ukey >= TOPBIT, ukey & LOWMASK, ~ukey)
    return lax.bitcast_convert_type(b, jnp.float32)


def _tc_body(params_ref, logits_ref, targets_ref, out_ref):
    rows = logits_ref.shape[0]
    nchunk = rows // CHUNK
    tpos = lax.bitcast_convert_type(params_ref[0, 0], jnp.uint32)
    rpos = params_ref[0, 1]
    tneg = lax.bitcast_convert_type(params_ref[1, 0], jnp.uint32)
    rneg = params_ref[1, 1]

    def fin_chunk(i, carry):
        s1, s2 = carry
        x = logits_ref[pl.ds(i * CHUNK, CHUNK), :]
        t = targets_ref[pl.ds(i * CHUNK, CHUNK), :]
        b = lax.bitcast_convert_type(x, jnp.uint32)
        ukey = jnp.where(b >= TOPBIT, ~b, b | TOPBIT)
        kpos = jnp.where(t >= 1, ~ukey, np.uint32(0))
        kneg = jnp.where(t == 0, ukey, np.uint32(0))
        l1 = jnp.minimum(_softplus(-x), 100.0)
        l2 = jnp.minimum(_softplus(x), 100.0)
        s1 = s1 + jnp.sum(jnp.where(kpos > tpos, l1, 0.0))
        s2 = s2 + jnp.sum(jnp.where(kneg > tneg, l2, 0.0))
        return s1, s2

    s1, s2 = lax.fori_loop(0, nchunk, fin_chunk, (0.0, 0.0))
    tl1v = jnp.minimum(_softplus(-_unmono(jnp.full((8, 128), ~tpos))), 100.0)
    tl2v = jnp.minimum(_softplus(_unmono(jnp.full((8, 128), tneg))), 100.0)
    i0 = lax.broadcasted_iota(jnp.int32, (8, 128), 0)
    i1 = lax.broadcasted_iota(jnp.int32, (8, 128), 1)
    m00 = (i0 == 0) & (i1 == 0)
    s1 = s1 + jnp.sum(jnp.where(m00 & (rpos > 0),
                                rpos.astype(jnp.float32) * tl1v, 0.0))
    s2 = s2 + jnp.sum(jnp.where(m00 & (rneg > 0),
                                rneg.astype(jnp.float32) * tl2v, 0.0))
    out_ref[...] = jnp.broadcast_to((s1 + s2) / float(BATCH), (1, 1))


N_TOTAL = 1048576
PER_TILE = N_TOTAL // 16


def _make_sck():
    mesh = plsc.VectorSubcoreMesh(core_axis_name="c", subcore_axis_name="s")
    return pl.kernel(
        _sc_body,
        mesh=mesh,
        out_type=jax.ShapeDtypeStruct((2, 16), jnp.int32),
        compiler_params=pltpu.CompilerParams(needs_layout_passes=False),
        scratch_types=[
            pltpu.VMEM((SC_CH,), jnp.float32),
            pltpu.VMEM((SC_CH,), jnp.int32),
            pltpu.VMEM((NBINS * 16,), jnp.int32),
            pltpu.VMEM((128, 16), jnp.int32),
            pltpu.VMEM((128, 16), jnp.int32),
            pltpu.VMEM((16,), jnp.int32),
            pltpu.VMEM_SHARED((16, 128, 16), jnp.int32),
        ],
    )


_SCK = _make_sck()


def kernel(logits, targets):
    n = logits.shape[0]
    flat = logits.reshape(n)
    params = _SCK(flat, targets)

    rows = n // LANES
    out = pl.pallas_call(
        _tc_body,
        out_shape=jax.ShapeDtypeStruct((1, 1), jnp.float32),
        in_specs=[
            pl.BlockSpec(memory_space=pltpu.SMEM),
            pl.BlockSpec(memory_space=pltpu.VMEM),
            pl.BlockSpec(memory_space=pltpu.VMEM),
        ],
        out_specs=pl.BlockSpec(memory_space=pltpu.VMEM),
    )(params, flat.reshape(rows, LANES), targets.reshape(rows, LANES))
    return out[0, 0]


# speedup vs baseline: 1.5160x; 1.3198x over previous
import functools

import jax
import jax.numpy as jnp
import numpy as np
from jax import lax
from jax.experimental import pallas as pl
from jax.experimental.pallas import tpu as pltpu
from jax.experimental.pallas import tpu_sc as plsc

LANES = 128
CHUNK = 512
KPOS_MAX = 128
BATCH = 512
TOPBIT = np.uint32(0x80000000)
LOWMASK = np.uint32(0x7FFFFFFF)
NBINS = 2048
SC_CH = 4096


def _sc_body(logits_hbm, targets_hbm, out_hbm, xbuf, tbuf, hist, red,
             mbuf, tmpv, xbuf2, tbuf2, sx0, st0, sx1, st1, shared2):
    cid = lax.axis_index("c")
    sid = lax.axis_index("s")
    per_tile = logits_hbm.shape[0] // 16
    nch = per_tile // SC_CH
    iota = lax.iota(jnp.int32, 16)
    zero16 = jnp.zeros((16,), jnp.int32)
    ones16 = jnp.full((16,), 1, jnp.int32)
    is_c0 = cid == 0
    c0v = jnp.broadcast_to(is_c0, (16,))
    flip = jnp.where(c0v, jnp.full((16,), np.uint32(0xFFFFFFFF), jnp.uint32),
                     jnp.zeros((16,), jnp.uint32))
    lanebase = iota * NBINS

    def zero_hist():
        def zh(i, _):
            for u in range(4):
                plsc.store_scatter(hist, [(i * 4 + u) * 16 + iota], zero16)
            return 0
        lax.fori_loop(0, (NBINS * 16) // 64, zh, 0)

    zero_hist()

    def start_chunk(ci, xb, tb, sx, st):
        base = sid * per_tile + ci * SC_CH
        pltpu.make_async_copy(logits_hbm.at[pl.ds(base, SC_CH)], xb, sx).start()
        pltpu.make_async_copy(targets_hbm.at[pl.ds(base, SC_CH)], tb, st).start()

    def wait_chunk(xb, tb, sx, st):
        pltpu.make_async_copy(logits_hbm.at[pl.ds(0, SC_CH)], xb, sx).wait()
        pltpu.make_async_copy(targets_hbm.at[pl.ds(0, SC_CH)], tb, st).wait()

    def stream_pass(binfn):
        def process(xb, tb):
            def vec_body(v, _c):
                for u in range(4):
                    off = (v * 4 + u) * 16
                    x = xb[pl.ds(off, 16)]
                    t = tb[pl.ds(off, 16)]
                    b = lax.bitcast_convert_type(x, jnp.uint32)
                    neg = b >= TOPBIT
                    ukey = jnp.where(neg, ~b, b | TOPBIT)
                    key = ukey ^ flip
                    sel = (t >= 1) == c0v
                    keym = jnp.where(sel, key, np.uint32(0))
                    bn, m = binfn(keym, sel)
                    plsc.addupdate_scatter(hist, [lanebase + bn], ones16,
                                           mask=m)
                return 0
            lax.fori_loop(0, SC_CH // 64, vec_body, 0)

        nh = nch // 2
        start_chunk(0, xbuf, tbuf, sx0, st0)

        def loop2(i, _):
            c0 = i * 2
            start_chunk(c0 + 1, xbuf2, tbuf2, sx1, st1)
            wait_chunk(xbuf, tbuf, sx0, st0)
            process(xbuf, tbuf)

            @pl.when(i < nh - 1)
            def _():
                start_chunk(c0 + 2, xbuf, tbuf, sx0, st0)

            wait_chunk(xbuf2, tbuf2, sx1, st1)
            process(xbuf2, tbuf2)
            return 0
        lax.fori_loop(0, nh, loop2, 0)

    def bin_l1(keym, sel):
        return lax.bitcast_convert_type(keym >> np.uint32(21), jnp.int32), sel
    stream_pass(bin_l1)

    def reduce_hist(nrows):
        def rb(r, _):
            rbase = r * 16
            acc = zero16
            for l in range(16):
                acc = acc + plsc.load_gather(hist, [l * NBINS + rbase + iota])
            red[r] = acc
            return 0
        lax.fori_loop(0, nrows, rb, 0)

    def merge_and_fetch(nrows):
        pltpu.sync_copy(red, shared2.at[sid])
        plsc.subcore_barrier()
        pltpu.sync_copy(shared2.at[0], red)

        def mj(j, _):
            pltpu.sync_copy(shared2.at[j], mbuf)

            def rr(r, _2):
                for u in range(4):
                    rw = r * 4 + u
                    red[rw] = red[rw] + mbuf[rw]
                return 0
            lax.fori_loop(0, nrows // 4, rr, 0)
            return 0
        lax.fori_loop(1, 16, mj, 0)
        plsc.subcore_barrier()

    def scan_rows(nrows, k):
        def sb(i, carry):
            cum, row_sel, cum_above, row_keep = carry
            r = nrows - 1 - i
            row = red[r]
            s = jnp.sum(row)
            hit = (row_sel < 0) & (cum + s >= k) & (k > 0)
            hitv = jnp.broadcast_to(hit, (16,))
            row_sel = jnp.where(hit, r, row_sel)
            cum_above = jnp.where(hit, cum, cum_above)
            row_keep = jnp.where(hitv, row, row_keep)
            return (cum + s, row_sel, cum_above, row_keep)

        cum, row_sel, cum_above, row_keep = lax.fori_loop(
            0, nrows, sb, (0, -1, 0, zero16))
        rv = jnp.flip(row_keep, 0)
        c = jnp.cumsum(rv)
        kr = k - cum_above
        j = jnp.min(jnp.where(c >= kr, iota, 16))
        jv = jnp.broadcast_to(j, (16,))
        cj = jnp.sum(jnp.where(iota == jv, c, 0))
        rj = jnp.sum(jnp.where(iota == jv, rv, 0))
        b = row_sel * 16 + (15 - j)
        above = cum_above + cj - rj
        return b, above, cum

    def refine_pass(mshift, mprefix, bshift, bmask, nrows):
        zero_hist()
        pm = jnp.broadcast_to(mprefix, (16,))

        def bin_ref(keym, sel):
            match = (keym >> mshift) == pm
            bn = lax.bitcast_convert_type((keym >> bshift) & bmask, jnp.int32)
            return bn, match
        stream_pass(bin_ref)
        reduce_hist(nrows)
        merge_and_fetch(nrows)

    reduce_hist(128)
    merge_and_fetch(128)
    n_total = per_tile * 16
    def tot_body(r, acc):
        return acc + jnp.sum(red[r])
    own_total = lax.fori_loop(0, 128, tot_body, 0)
    cpos = jnp.where(is_c0, own_total, n_total - own_total)
    cneg = n_total - cpos
    k1 = jnp.minimum(cpos, KPOS_MAX)
    k2 = jnp.minimum(BATCH - k1, cneg)
    k = jnp.where(is_c0, k1, k2)

    b1, a1, _ = scan_rows(128, k)
    p1u = b1.astype(jnp.uint32)

    refine_pass(np.uint32(21), p1u, np.uint32(10), np.uint32(0x7FF), 128)
    b2, a2, _ = scan_rows(128, k - a1)
    p2u = (p1u << np.uint32(11)) | b2.astype(jnp.uint32)

    refine_pass(np.uint32(10), p2u, np.uint32(0), np.uint32(0x3FF), 64)
    b3, a3, _ = scan_rows(64, k - a1 - a2)

    t_key = (p2u << np.uint32(10)) | b3.astype(jnp.uint32)
    t_key = jnp.where(k > 0, t_key, np.uint32(0xFFFFFFFF))
    r_tie = jnp.where(k > 0, k - (a1 + a2 + a3), 0)

    tbits = lax.bitcast_convert_type(jnp.broadcast_to(t_key, (16,)), jnp.int32)
    outv = jnp.where(iota == 0, tbits, zero16)
    outv = jnp.where(iota == 1, jnp.broadcast_to(r_tie, (16,)), outv)
    outv = jnp.where(iota == 2, jnp.broadcast_to(cpos, (16,)), outv)
    outv = jnp.where(iota == 3, jnp.broadcast_to(cneg, (16,)), outv)

    @pl.when(sid == 0)
    def _():
        tmpv[...] = outv
        pltpu.sync_copy(tmpv, out_hbm.at[cid])


def _softplus(t):
    return jnp.maximum(t, 0.0) + jnp.log1p(jnp.exp(-jnp.abs(t)))


def _unmono(ukey):
    b = jnp.where(ukey >= TOPBIT, ukey & LOWMASK, ~ukey)
    return lax.bitcast_convert_type(b, jnp.float32)


def _tc_body(params_ref, logits_ref, targets_ref, out_ref):
    rows = logits_ref.shape[0]
    nchunk = rows // CHUNK
    tpos = lax.bitcast_convert_type(params_ref[0, 0], jnp.uint32)
    rpos = params_ref[0, 1]
    tneg = lax.bitcast_convert_type(params_ref[1, 0], jnp.uint32)
    rneg = params_ref[1, 1]

    def fin_chunk(i, carry):
        s1, s2 = carry
        x = logits_ref[pl.ds(i * CHUNK, CHUNK), :]
        t = targets_ref[pl.ds(i * CHUNK, CHUNK), :]
        b = lax.bitcast_convert_type(x, jnp.uint32)
        ukey = jnp.where(b >= TOPBIT, ~b, b | TOPBIT)
        kpos = jnp.where(t >= 1, ~ukey, np.uint32(0))
        kneg = jnp.where(t == 0, ukey, np.uint32(0))
        l1 = jnp.minimum(_softplus(-x), 100.0)
        l2 = jnp.minimum(_softplus(x), 100.0)
        s1 = s1 + jnp.sum(jnp.where(kpos > tpos, l1, 0.0))
        s2 = s2 + jnp.sum(jnp.where(kneg > tneg, l2, 0.0))
        return s1, s2

    s1, s2 = lax.fori_loop(0, nchunk, fin_chunk, (0.0, 0.0))
    tl1v = jnp.minimum(_softplus(-_unmono(jnp.full((8, 128), ~tpos))), 100.0)
    tl2v = jnp.minimum(_softplus(_unmono(jnp.full((8, 128), tneg))), 100.0)
    i0 = lax.broadcasted_iota(jnp.int32, (8, 128), 0)
    i1 = lax.broadcasted_iota(jnp.int32, (8, 128), 1)
    m00 = (i0 == 0) & (i1 == 0)
    s1 = s1 + jnp.sum(jnp.where(m00 & (rpos > 0),
                                rpos.astype(jnp.float32) * tl1v, 0.0))
    s2 = s2 + jnp.sum(jnp.where(m00 & (rneg > 0),
                                rneg.astype(jnp.float32) * tl2v, 0.0))
    out_ref[...] = jnp.broadcast_to((s1 + s2) / float(BATCH), (1, 1))


N_TOTAL = 1048576
PER_TILE = N_TOTAL // 16


def _make_sck():
    mesh = plsc.VectorSubcoreMesh(core_axis_name="c", subcore_axis_name="s")
    return pl.kernel(
        _sc_body,
        mesh=mesh,
        out_type=jax.ShapeDtypeStruct((2, 16), jnp.int32),
        compiler_params=pltpu.CompilerParams(needs_layout_passes=False),
        scratch_types=[
            pltpu.VMEM((SC_CH,), jnp.float32),
            pltpu.VMEM((SC_CH,), jnp.int32),
            pltpu.VMEM((NBINS * 16,), jnp.int32),
            pltpu.VMEM((128, 16), jnp.int32),
            pltpu.VMEM((128, 16), jnp.int32),
            pltpu.VMEM((16,), jnp.int32),
            pltpu.VMEM((SC_CH,), jnp.float32),
            pltpu.VMEM((SC_CH,), jnp.int32),
            pltpu.SemaphoreType.DMA,
            pltpu.SemaphoreType.DMA,
            pltpu.SemaphoreType.DMA,
            pltpu.SemaphoreType.DMA,
            pltpu.VMEM_SHARED((16, 128, 16), jnp.int32),
        ],
    )


_SCK = _make_sck()


def kernel(logits, targets):
    n = logits.shape[0]
    flat = logits.reshape(n)
    params = _SCK(flat, targets)

    rows = n // LANES
    out = pl.pallas_call(
        _tc_body,
        out_shape=jax.ShapeDtypeStruct((1, 1), jnp.float32),
        in_specs=[
            pl.BlockSpec(memory_space=pltpu.SMEM),
            pl.BlockSpec(memory_space=pltpu.VMEM),
            pl.BlockSpec(memory_space=pltpu.VMEM),
        ],
        out_specs=pl.BlockSpec(memory_space=pltpu.VMEM),
    )(params, flat.reshape(rows, LANES), targets.reshape(rows, LANES))
    return out[0, 0]
